# Initial kernel scaffold; baseline (speedup 1.0000x reference)
import jax
import jax.numpy as jnp
from jax.experimental import pallas as pl


def kernel(X):
    raise NotImplementedError("write your pallas kernel here")


# speedup vs baseline: 1.0406x; 1.0406x over previous
import jax
import jax.numpy as jnp
from jax import lax
from jax.experimental import pallas as pl
from jax.experimental.pallas import tpu as pltpu

_N_CLUSTERS = 512
_MAX_ITER = 5
_BLK = 2048


def _init_centroids(X, n_clusters, key):
    n = X.shape[0]
    k0, kseq = jax.random.split(key)
    idx0 = jax.random.randint(k0, (), 0, n)
    c0 = X[idx0]
    closest = jnp.sum((X - c0[None, :]) ** 2, axis=1)
    keys = jax.random.split(kseq, n_clusters - 1)

    def step(closest, sub):
        probs = closest / (closest.sum() + 1e-10)
        idx = jax.random.choice(sub, n, p=probs)
        newc = X[idx]
        d = jnp.sum((X - newc[None, :]) ** 2, axis=1)
        return jnp.minimum(closest, d), newc

    _, rest = lax.scan(step, closest, keys)
    return jnp.concatenate([c0[None, :], rest], axis=0)


def _loop_kernel(x_ref, c0t_ref, replt_ref, labels_ref, centt_ref, inert_ref,
                 ct_scr, sumst_scr, counts_scr):
    N, D = x_ref.shape
    K = c0t_ref.shape[1]
    nblk = N // _BLK
    ct_scr[...] = c0t_ref[...]

    def iter_body(it, carry):
        CT = ct_scr[...]
        c2 = jnp.sum(CT * CT, axis=0, keepdims=True)
        sumst_scr[...] = jnp.zeros((D, K), jnp.float32)
        counts_scr[...] = jnp.zeros((1, K), jnp.float32)

        def blk_body(b, acc):
            xb = x_ref[pl.ds(b * _BLK, _BLK), :]
            s = lax.dot_general(xb, CT, (((1,), (0,)), ((), ())),
                                preferred_element_type=jnp.float32)
            x2 = jnp.sum(xb * xb, axis=1, keepdims=True)
            d2 = jnp.maximum(x2 + c2 - 2.0 * s, 0.0)
            lab = jnp.argmin(d2, axis=1).astype(jnp.int32)
            labels_ref[pl.ds(b * _BLK, _BLK)] = lab
            oh = (lab[:, None] ==
                  lax.broadcasted_iota(jnp.int32, (_BLK, K), 1)
                  ).astype(jnp.float32)
            sumst_scr[...] += lax.dot_general(
                xb, oh, (((0,), (0,)), ((), ())),
                precision=lax.Precision.HIGHEST,
                preferred_element_type=jnp.float32)
            counts_scr[...] += jnp.sum(oh, axis=0, keepdims=True)
            return acc + jnp.sum(jnp.min(d2, axis=1))

        inert = lax.fori_loop(0, nblk, blk_body, jnp.float32(0.0))
        inert_ref[...] = inert.reshape(1, 1)
        counts = counts_scr[...]
        meansT = sumst_scr[...] / jnp.maximum(counts, 1.0)
        ct_scr[...] = jnp.where(counts > 0, meansT, replt_ref[it])
        return carry

    lax.fori_loop(0, _MAX_ITER, iter_body, jnp.int32(0))
    centt_ref[...] = ct_scr[...]


def _kmeans_loop(X, C0, repl):
    N, D = X.shape
    K = C0.shape[0]
    labels, centT, inert = pl.pallas_call(
        _loop_kernel,
        out_shape=(
            jax.ShapeDtypeStruct((N,), jnp.int32),
            jax.ShapeDtypeStruct((D, K), jnp.float32),
            jax.ShapeDtypeStruct((1, 1), jnp.float32),
        ),
        scratch_shapes=[
            pltpu.VMEM((D, K), jnp.float32),
            pltpu.VMEM((D, K), jnp.float32),
            pltpu.VMEM((1, K), jnp.float32),
        ],
    )(X, C0.T, jnp.transpose(repl, (0, 2, 1)))
    return labels, centT.T, inert[0, 0]


def kernel(X):
    n = X.shape[0]
    key = jax.random.key(42)
    kinit, kloop = jax.random.split(key)
    C0 = _init_centroids(X, _N_CLUSTERS, kinit)
    repl = jnp.stack([
        X[jax.random.randint(jax.random.fold_in(kloop, i),
                             (_N_CLUSTERS,), 0, n)]
        for i in range(_MAX_ITER)
    ])
    labels, cent, inert = _kmeans_loop(X, C0, repl)
    return labels, cent, inert
